# baseline (device time: 1066954 ns/iter reference)
import jax
import jax.numpy as jnp
from jax import lax
from jax.experimental import pallas as pl
from jax.experimental.pallas import tpu as pltpu

N_CHUNKS = 8


def kernel(x):
    m_per, n = x.shape
    half = m_per // 2
    ch = half // N_CHUNKS

    def body(x_ref, out_ref, copy_sem, y_send, y_recv, x_send, x_recv):
        my_x = lax.axis_index("x")
        my_y = lax.axis_index("y")
        other_x = 1 - my_x
        other_y = 1 - my_y

        barrier_sem = pltpu.get_barrier_semaphore()
        for nbr in ((my_x, other_y), (other_x, my_y)):
            pl.semaphore_signal(
                barrier_sem,
                inc=1,
                device_id=nbr,
                device_id_type=pl.DeviceIdType.MESH,
            )
        pl.semaphore_wait(barrier_sem, 2)

        local = pltpu.make_async_copy(
            x_ref, out_ref.at[pl.ds(my_y * m_per, m_per), :], copy_sem
        )
        local.start()

        y_rdmas = []
        for c in range(N_CHUNKS):
            off = my_x * half + c * ch
            rdma = pltpu.make_async_remote_copy(
                src_ref=x_ref.at[pl.ds(off, ch), :],
                dst_ref=out_ref.at[pl.ds(my_y * m_per + off, ch), :],
                send_sem=y_send.at[c],
                recv_sem=y_recv.at[c],
                device_id=(my_x, other_y),
                device_id_type=pl.DeviceIdType.MESH,
            )
            rdma.start()
            y_rdmas.append(rdma)

        x_rdmas = []
        for c in range(N_CHUNKS):
            y_rdmas[c].wait_recv()
            off = other_y * m_per + my_x * half + c * ch
            fwd = pltpu.make_async_remote_copy(
                src_ref=out_ref.at[pl.ds(off, ch), :],
                dst_ref=out_ref.at[pl.ds(off, ch), :],
                send_sem=x_send.at[c],
                recv_sem=x_recv.at[c],
                device_id=(other_x, my_y),
                device_id_type=pl.DeviceIdType.MESH,
            )
            fwd.start()
            x_rdmas.append(fwd)

        for c in range(N_CHUNKS):
            x_rdmas[c].wait_recv()
        for c in range(N_CHUNKS):
            y_rdmas[c].wait_send()
            x_rdmas[c].wait_send()
        local.wait()

    return pl.pallas_call(
        body,
        out_shape=jax.ShapeDtypeStruct((2 * m_per, n), x.dtype),
        in_specs=[pl.BlockSpec(memory_space=pl.ANY)],
        out_specs=pl.BlockSpec(memory_space=pl.ANY),
        scratch_shapes=[
            pltpu.SemaphoreType.DMA,
            pltpu.SemaphoreType.DMA((N_CHUNKS,)),
            pltpu.SemaphoreType.DMA((N_CHUNKS,)),
            pltpu.SemaphoreType.DMA((N_CHUNKS,)),
            pltpu.SemaphoreType.DMA((N_CHUNKS,)),
        ],
        compiler_params=pltpu.CompilerParams(collective_id=0),
    )(x)


# device time: 244147 ns/iter; 4.3701x vs baseline; 4.3701x over previous
import jax
import jax.numpy as jnp
from jax import lax
from jax.experimental import pallas as pl
from jax.experimental.pallas import tpu as pltpu

N_CHUNKS = 16
K_LOCAL = 8


def kernel(x):
    m_per, n = x.shape
    half = m_per // 2
    ch = half // N_CHUNKS
    rows_l = m_per // K_LOCAL

    def body(x_ref, out_ref, vbuf, y_send, y_recv, x_send, x_recv,
             in_sems, out_sems):
        my_x = lax.axis_index("x")
        my_y = lax.axis_index("y")
        other_x = 1 - my_x
        other_y = 1 - my_y

        barrier_sem = pltpu.get_barrier_semaphore()
        for nbr in ((my_x, other_y), (other_x, my_y)):
            pl.semaphore_signal(
                barrier_sem,
                inc=1,
                device_id=nbr,
                device_id_type=pl.DeviceIdType.MESH,
            )
        pl.semaphore_wait(barrier_sem, 2)

        y_rdmas = []
        for c in range(N_CHUNKS):
            off = my_x * half + c * ch
            rdma = pltpu.make_async_remote_copy(
                src_ref=x_ref.at[pl.ds(off, ch), :],
                dst_ref=out_ref.at[pl.ds(my_y * m_per + off, ch), :],
                send_sem=y_send.at[c],
                recv_sem=y_recv.at[c],
                device_id=(my_x, other_y),
                device_id_type=pl.DeviceIdType.MESH,
            )
            rdma.start()
            y_rdmas.append(rdma)

        lins = [
            pltpu.make_async_copy(
                x_ref.at[pl.ds(k * rows_l, rows_l), :],
                vbuf.at[k % 2],
                in_sems.at[k % 2],
            )
            for k in range(K_LOCAL)
        ]
        louts = [None] * K_LOCAL
        lins[0].start()

        x_rdmas = []
        for c in range(N_CHUNKS):
            y_rdmas[c].wait_recv()
            off = other_y * m_per + my_x * half + c * ch
            fwd = pltpu.make_async_remote_copy(
                src_ref=out_ref.at[pl.ds(off, ch), :],
                dst_ref=out_ref.at[pl.ds(off, ch), :],
                send_sem=x_send.at[c],
                recv_sem=x_recv.at[c],
                device_id=(other_x, my_y),
                device_id_type=pl.DeviceIdType.MESH,
            )
            fwd.start()
            x_rdmas.append(fwd)

            if c < K_LOCAL:
                k = c
                lins[k].wait()
                if k >= 1:
                    louts[k - 1].wait()
                if k + 1 < K_LOCAL:
                    lins[k + 1].start()
                lout = pltpu.make_async_copy(
                    vbuf.at[k % 2],
                    out_ref.at[pl.ds(my_y * m_per + k * rows_l, rows_l), :],
                    out_sems.at[k % 2],
                )
                lout.start()
                louts[k] = lout

        for c in range(N_CHUNKS):
            x_rdmas[c].wait_recv()
        for c in range(N_CHUNKS):
            y_rdmas[c].wait_send()
            x_rdmas[c].wait_send()
        louts[K_LOCAL - 1].wait()

    return pl.pallas_call(
        body,
        out_shape=jax.ShapeDtypeStruct((2 * m_per, n), x.dtype),
        in_specs=[pl.BlockSpec(memory_space=pl.ANY)],
        out_specs=pl.BlockSpec(memory_space=pl.ANY),
        scratch_shapes=[
            pltpu.VMEM((2, rows_l, n), x.dtype),
            pltpu.SemaphoreType.DMA((N_CHUNKS,)),
            pltpu.SemaphoreType.DMA((N_CHUNKS,)),
            pltpu.SemaphoreType.DMA((N_CHUNKS,)),
            pltpu.SemaphoreType.DMA((N_CHUNKS,)),
            pltpu.SemaphoreType.DMA((2,)),
            pltpu.SemaphoreType.DMA((2,)),
        ],
        compiler_params=pltpu.CompilerParams(collective_id=0),
    )(x)


# device time: 242685 ns/iter; 4.3965x vs baseline; 1.0060x over previous
import jax
import jax.numpy as jnp
from jax import lax
from jax.experimental import pallas as pl
from jax.experimental.pallas import tpu as pltpu

N_CHUNKS = 16
K_LOCAL = 8


def kernel(x):
    m_per, n = x.shape
    half = m_per // 2
    ch = half // N_CHUNKS
    rows_l = m_per // K_LOCAL

    def body(x_ref, out_ref, ybuf, vbuf, y_send, y_recv, x_send, x_recv,
             drain_sems, in_sems, out_sems):
        my_x = lax.axis_index("x")
        my_y = lax.axis_index("y")
        other_x = 1 - my_x
        other_y = 1 - my_y

        barrier_sem = pltpu.get_barrier_semaphore()
        for nbr in ((my_x, other_y), (other_x, my_y)):
            pl.semaphore_signal(
                barrier_sem,
                inc=1,
                device_id=nbr,
                device_id_type=pl.DeviceIdType.MESH,
            )
        pl.semaphore_wait(barrier_sem, 2)

        y_rdmas = []
        for c in range(N_CHUNKS):
            off = my_x * half + c * ch
            rdma = pltpu.make_async_remote_copy(
                src_ref=x_ref.at[pl.ds(off, ch), :],
                dst_ref=ybuf.at[c],
                send_sem=y_send.at[c],
                recv_sem=y_recv.at[c],
                device_id=(my_x, other_y),
                device_id_type=pl.DeviceIdType.MESH,
            )
            rdma.start()
            y_rdmas.append(rdma)

        lins = [
            pltpu.make_async_copy(
                x_ref.at[pl.ds(k * rows_l, rows_l), :],
                vbuf.at[k % 2],
                in_sems.at[k % 2],
            )
            for k in range(K_LOCAL)
        ]
        louts = [None] * K_LOCAL
        lins[0].start()

        x_rdmas = []
        for c in range(N_CHUNKS):
            y_rdmas[c].wait_recv()
            off = other_y * m_per + my_x * half + c * ch
            fwd = pltpu.make_async_remote_copy(
                src_ref=ybuf.at[c],
                dst_ref=out_ref.at[pl.ds(off, ch), :],
                send_sem=x_send.at[c],
                recv_sem=x_recv.at[c],
                device_id=(other_x, my_y),
                device_id_type=pl.DeviceIdType.MESH,
            )
            fwd.start()
            x_rdmas.append(fwd)
            drain = pltpu.make_async_copy(
                ybuf.at[c],
                out_ref.at[pl.ds(off, ch), :],
                drain_sems.at[c],
            )
            drain.start()

            if c < K_LOCAL:
                k = c
                lins[k].wait()
                if k >= 1:
                    louts[k - 1].wait()
                if k + 1 < K_LOCAL:
                    lins[k + 1].start()
                lout = pltpu.make_async_copy(
                    vbuf.at[k % 2],
                    out_ref.at[pl.ds(my_y * m_per + k * rows_l, rows_l), :],
                    out_sems.at[k % 2],
                )
                lout.start()
                louts[k] = lout

        for c in range(N_CHUNKS):
            x_rdmas[c].wait_recv()
        for c in range(N_CHUNKS):
            y_rdmas[c].wait_send()
            x_rdmas[c].wait_send()
            pltpu.make_async_copy(
                ybuf.at[c],
                out_ref.at[pl.ds(other_y * m_per + my_x * half + c * ch, ch), :],
                drain_sems.at[c],
            ).wait()
        louts[K_LOCAL - 1].wait()

    return pl.pallas_call(
        body,
        out_shape=jax.ShapeDtypeStruct((2 * m_per, n), x.dtype),
        in_specs=[pl.BlockSpec(memory_space=pl.ANY)],
        out_specs=pl.BlockSpec(memory_space=pl.ANY),
        scratch_shapes=[
            pltpu.VMEM((N_CHUNKS, ch, n), x.dtype),
            pltpu.VMEM((2, rows_l, n), x.dtype),
            pltpu.SemaphoreType.DMA((N_CHUNKS,)),
            pltpu.SemaphoreType.DMA((N_CHUNKS,)),
            pltpu.SemaphoreType.DMA((N_CHUNKS,)),
            pltpu.SemaphoreType.DMA((N_CHUNKS,)),
            pltpu.SemaphoreType.DMA((N_CHUNKS,)),
            pltpu.SemaphoreType.DMA((2,)),
            pltpu.SemaphoreType.DMA((2,)),
        ],
        compiler_params=pltpu.CompilerParams(collective_id=0),
    )(x)


# device time: 238102 ns/iter; 4.4811x vs baseline; 1.0192x over previous
import jax
import jax.numpy as jnp
from jax import lax
from jax.experimental import pallas as pl
from jax.experimental.pallas import tpu as pltpu

N_CHUNKS = 32
K_LOCAL = 8


def kernel(x):
    m_per, n = x.shape
    half = m_per // 2
    ch = half // N_CHUNKS
    rows_l = m_per // K_LOCAL

    def body(x_ref, out_ref, ybuf, vbuf, y_send, y_recv, x_send, x_recv,
             drain_sems, in_sems, out_sems):
        my_x = lax.axis_index("x")
        my_y = lax.axis_index("y")
        other_x = 1 - my_x
        other_y = 1 - my_y

        barrier_sem = pltpu.get_barrier_semaphore()
        for nbr in ((my_x, other_y), (other_x, my_y)):
            pl.semaphore_signal(
                barrier_sem,
                inc=1,
                device_id=nbr,
                device_id_type=pl.DeviceIdType.MESH,
            )
        pl.semaphore_wait(barrier_sem, 2)

        y_rdmas = []
        for c in range(N_CHUNKS):
            off = my_x * half + c * ch
            rdma = pltpu.make_async_remote_copy(
                src_ref=x_ref.at[pl.ds(off, ch), :],
                dst_ref=ybuf.at[c],
                send_sem=y_send.at[c],
                recv_sem=y_recv.at[c],
                device_id=(my_x, other_y),
                device_id_type=pl.DeviceIdType.MESH,
            )
            rdma.start()
            y_rdmas.append(rdma)

        lins = [
            pltpu.make_async_copy(
                x_ref.at[pl.ds(k * rows_l, rows_l), :],
                vbuf.at[k % 2],
                in_sems.at[k % 2],
            )
            for k in range(K_LOCAL)
        ]
        louts = [None] * K_LOCAL
        lins[0].start()

        x_rdmas = []
        for c in range(N_CHUNKS):
            y_rdmas[c].wait_recv()
            off = other_y * m_per + my_x * half + c * ch
            fwd = pltpu.make_async_remote_copy(
                src_ref=ybuf.at[c],
                dst_ref=out_ref.at[pl.ds(off, ch), :],
                send_sem=x_send.at[c],
                recv_sem=x_recv.at[c],
                device_id=(other_x, my_y),
                device_id_type=pl.DeviceIdType.MESH,
            )
            fwd.start()
            x_rdmas.append(fwd)
            drain = pltpu.make_async_copy(
                ybuf.at[c],
                out_ref.at[pl.ds(off, ch), :],
                drain_sems.at[c],
            )
            drain.start()

            if c < K_LOCAL:
                k = c
                lins[k].wait()
                if k >= 1:
                    louts[k - 1].wait()
                if k + 1 < K_LOCAL:
                    lins[k + 1].start()
                lout = pltpu.make_async_copy(
                    vbuf.at[k % 2],
                    out_ref.at[pl.ds(my_y * m_per + k * rows_l, rows_l), :],
                    out_sems.at[k % 2],
                )
                lout.start()
                louts[k] = lout

        for c in range(N_CHUNKS):
            x_rdmas[c].wait_recv()
        for c in range(N_CHUNKS):
            y_rdmas[c].wait_send()
            x_rdmas[c].wait_send()
            pltpu.make_async_copy(
                ybuf.at[c],
                out_ref.at[pl.ds(other_y * m_per + my_x * half + c * ch, ch), :],
                drain_sems.at[c],
            ).wait()
        louts[K_LOCAL - 1].wait()

    return pl.pallas_call(
        body,
        out_shape=jax.ShapeDtypeStruct((2 * m_per, n), x.dtype),
        in_specs=[pl.BlockSpec(memory_space=pl.ANY)],
        out_specs=pl.BlockSpec(memory_space=pl.ANY),
        scratch_shapes=[
            pltpu.VMEM((N_CHUNKS, ch, n), x.dtype),
            pltpu.VMEM((2, rows_l, n), x.dtype),
            pltpu.SemaphoreType.DMA((N_CHUNKS,)),
            pltpu.SemaphoreType.DMA((N_CHUNKS,)),
            pltpu.SemaphoreType.DMA((N_CHUNKS,)),
            pltpu.SemaphoreType.DMA((N_CHUNKS,)),
            pltpu.SemaphoreType.DMA((N_CHUNKS,)),
            pltpu.SemaphoreType.DMA((2,)),
            pltpu.SemaphoreType.DMA((2,)),
        ],
        compiler_params=pltpu.CompilerParams(collective_id=0),
    )(x)
